# baseline (speedup 1.0000x reference)
import functools

import jax
import jax.numpy as jnp
from jax import lax
from jax.experimental import pallas as pl
from jax.experimental.pallas import tpu as pltpu
from jax.experimental.pallas import tpu_sc as plsc

BATCH = 16384
DIM_MLP = 32
DIM_GMF = 16
EPS = 1e-5

NUM_CORES = 2
NUM_SUBCORES = 16
NUM_WORKERS = NUM_CORES * NUM_SUBCORES
B_PER_W = BATCH // NUM_WORKERS


def _sc_gather_body(u_idx_hbm, i_idx_hbm, eu_mlp, ei_mlp, eu_mf, ei_mf,
                    u_mlp_out, i_mlp_out, u_mf_out, i_mf_out,
                    uidx_v, iidx_v, um_v, im_v, uf_v, if_v,
                    sem0, sem1, sem2, sem3):
    wid = lax.axis_index("s") * NUM_CORES + lax.axis_index("c")
    base = wid * B_PER_W
    sl = pl.ds(base, B_PER_W)
    pltpu.sync_copy(u_idx_hbm.at[sl], uidx_v)
    pltpu.sync_copy(i_idx_hbm.at[sl], iidx_v)
    c0 = pltpu.async_copy(eu_mlp.at[uidx_v], um_v, sem0)
    c1 = pltpu.async_copy(ei_mlp.at[iidx_v], im_v, sem1)
    c2 = pltpu.async_copy(eu_mf.at[uidx_v], uf_v, sem2)
    c3 = pltpu.async_copy(ei_mf.at[iidx_v], if_v, sem3)
    c0.wait()
    pltpu.sync_copy(um_v, u_mlp_out.at[sl])
    c1.wait()
    pltpu.sync_copy(im_v, i_mlp_out.at[sl])
    c2.wait()
    pltpu.sync_copy(uf_v, u_mf_out.at[sl])
    c3.wait()
    pltpu.sync_copy(if_v, i_mf_out.at[sl])


def _make_sc_gather():
    return functools.partial(
        pl.kernel,
        out_type=(
            jax.ShapeDtypeStruct((BATCH, DIM_MLP), jnp.float32),
            jax.ShapeDtypeStruct((BATCH, DIM_MLP), jnp.float32),
            jax.ShapeDtypeStruct((BATCH, DIM_GMF), jnp.float32),
            jax.ShapeDtypeStruct((BATCH, DIM_GMF), jnp.float32),
        ),
        mesh=plsc.VectorSubcoreMesh(core_axis_name="c", subcore_axis_name="s"),
        scratch_types=[
            pltpu.VMEM((B_PER_W,), jnp.int32),
            pltpu.VMEM((B_PER_W,), jnp.int32),
            pltpu.VMEM((B_PER_W, DIM_MLP), jnp.float32),
            pltpu.VMEM((B_PER_W, DIM_MLP), jnp.float32),
            pltpu.VMEM((B_PER_W, DIM_GMF), jnp.float32),
            pltpu.VMEM((B_PER_W, DIM_GMF), jnp.float32),
            pltpu.SemaphoreType.DMA,
            pltpu.SemaphoreType.DMA,
            pltpu.SemaphoreType.DMA,
            pltpu.SemaphoreType.DMA,
        ],
        compiler_params=pltpu.CompilerParams(use_tc_tiling_on_sc=False),
    )(_sc_gather_body)


def _tc_mlp_body(um, im, uf, if_, w1, b1, g1, be1, w2, b2, g2, be2, wout, out):
    w1_ = w1[...]
    nt = (((1,), (1,)), ((), ()))
    h = lax.dot_general(um[...], w1_[:, :DIM_MLP], nt,
                        preferred_element_type=jnp.float32)
    h = h + lax.dot_general(im[...], w1_[:, DIM_MLP:], nt,
                            preferred_element_type=jnp.float32)
    h = jnp.maximum(h + b1[...], 0.0)
    mean = jnp.mean(h, axis=0, keepdims=True)
    var = jnp.mean((h - mean) ** 2, axis=0, keepdims=True)
    h = (h - mean) * lax.rsqrt(var + EPS) * g1[...] + be1[...]
    h = lax.dot_general(h, w2[...], nt, preferred_element_type=jnp.float32)
    h = jnp.maximum(h + b2[...], 0.0)
    mean = jnp.mean(h, axis=0, keepdims=True)
    var = jnp.mean((h - mean) ** 2, axis=0, keepdims=True)
    h = (h - mean) * lax.rsqrt(var + EPS) * g2[...] + be2[...]
    mf = uf[...] * if_[...]
    wout_ = wout[...]
    logits = (jnp.sum(h * wout_[:, :16], axis=1, keepdims=True)
              + jnp.sum(mf * wout_[:, 16:], axis=1, keepdims=True))
    out[...] = jax.nn.sigmoid(logits)


def kernel(user_indices, item_indices, E_user_mlp, E_item_mlp, E_user_mf,
           E_item_mf, W1, b1, g1, be1, W2, b2, g2, be2, Wout):
    u_idx = user_indices.astype(jnp.int32)
    i_idx = item_indices.astype(jnp.int32)
    um, im, uf, if_ = _make_sc_gather()(u_idx, i_idx, E_user_mlp, E_item_mlp,
                                        E_user_mf, E_item_mf)
    out = pl.pallas_call(
        _tc_mlp_body,
        out_shape=jax.ShapeDtypeStruct((BATCH, 1), jnp.float32),
    )(um, im, uf, if_,
      W1, b1.reshape(1, -1), g1.reshape(1, -1), be1.reshape(1, -1),
      W2, b2.reshape(1, -1), g2.reshape(1, -1), be2.reshape(1, -1), Wout)
    return out


# speedup vs baseline: 1.2634x; 1.2634x over previous
import functools

import jax
import jax.numpy as jnp
from jax import lax
from jax.experimental import pallas as pl
from jax.experimental.pallas import tpu as pltpu
from jax.experimental.pallas import tpu_sc as plsc

BATCH = 16384
NUM_ROWS = 1000000
DIM_MLP = 32
DIM_GMF = 16
EPS = 1e-5

NUM_CORES = 2
NUM_SUBCORES = 16
NUM_WORKERS = NUM_CORES * NUM_SUBCORES
B_PER_W = BATCH // NUM_WORKERS

CHUNK = 8192
GRID = (NUM_ROWS + CHUNK - 1) // CHUNK
PK_MLP_ROWS = NUM_ROWS * DIM_MLP // 128
PK_GMF_ROWS = NUM_ROWS * DIM_GMF // 128


def _repack_one(src, dst, dim):
    rows_per = 128 // dim
    y = src[...].T.reshape(CHUNK // rows_per, rows_per, dim)
    for p in range(rows_per):
        dst[:, dim * p:dim * (p + 1)] = y[:, p, :]


def _repack_body(eu_mlp_t, ei_mlp_t, eu_mf_t, ei_mf_t,
                 pu_mlp, pi_mlp, pu_mf, pi_mf):
    _repack_one(eu_mlp_t, pu_mlp, DIM_MLP)
    _repack_one(ei_mlp_t, pi_mlp, DIM_MLP)
    _repack_one(eu_mf_t, pu_mf, DIM_GMF)
    _repack_one(ei_mf_t, pi_mf, DIM_GMF)


def _repack(eu_mlp_t, ei_mlp_t, eu_mf_t, ei_mf_t):
    mlp_in = pl.BlockSpec((DIM_MLP, CHUNK), lambda g: (0, g))
    mf_in = pl.BlockSpec((DIM_GMF, CHUNK), lambda g: (0, g))
    mlp_out = pl.BlockSpec((CHUNK * DIM_MLP // 128, 128), lambda g: (g, 0))
    mf_out = pl.BlockSpec((CHUNK * DIM_GMF // 128, 128), lambda g: (g, 0))
    return pl.pallas_call(
        _repack_body,
        grid=(GRID,),
        in_specs=[mlp_in, mlp_in, mf_in, mf_in],
        out_specs=[mlp_out, mlp_out, mf_out, mf_out],
        out_shape=[
            jax.ShapeDtypeStruct((PK_MLP_ROWS, 128), jnp.float32),
            jax.ShapeDtypeStruct((PK_MLP_ROWS, 128), jnp.float32),
            jax.ShapeDtypeStruct((PK_GMF_ROWS, 128), jnp.float32),
            jax.ShapeDtypeStruct((PK_GMF_ROWS, 128), jnp.float32),
        ],
    )(eu_mlp_t, ei_mlp_t, eu_mf_t, ei_mf_t)


def _sc_gather_body(u_idx_hbm, i_idx_hbm, eu_mlp, ei_mlp, eu_mf, ei_mf,
                    u_mlp_out, i_mlp_out, u_mf_out, i_mf_out,
                    uidx_v, iidx_v, um_v, im_v, uf_v, if_v,
                    sem0, sem1, sem2, sem3):
    wid = lax.axis_index("s") * NUM_CORES + lax.axis_index("c")
    base = wid * B_PER_W
    sl = pl.ds(base, B_PER_W)
    pltpu.sync_copy(u_idx_hbm.at[sl], uidx_v)
    pltpu.sync_copy(i_idx_hbm.at[sl], iidx_v)
    c0 = pltpu.async_copy(eu_mlp.at[uidx_v], um_v, sem0)
    c1 = pltpu.async_copy(ei_mlp.at[iidx_v], im_v, sem1)
    c2 = pltpu.async_copy(eu_mf.at[uidx_v], uf_v, sem2)
    c3 = pltpu.async_copy(ei_mf.at[iidx_v], if_v, sem3)
    c0.wait()
    pltpu.sync_copy(um_v, u_mlp_out.at[sl])
    c1.wait()
    pltpu.sync_copy(im_v, i_mlp_out.at[sl])
    c2.wait()
    pltpu.sync_copy(uf_v, u_mf_out.at[sl])
    c3.wait()
    pltpu.sync_copy(if_v, i_mf_out.at[sl])


def _make_sc_gather():
    return functools.partial(
        pl.kernel,
        out_type=(
            jax.ShapeDtypeStruct((BATCH, DIM_MLP), jnp.float32),
            jax.ShapeDtypeStruct((BATCH, DIM_MLP), jnp.float32),
            jax.ShapeDtypeStruct((BATCH, DIM_GMF), jnp.float32),
            jax.ShapeDtypeStruct((BATCH, DIM_GMF), jnp.float32),
        ),
        mesh=plsc.VectorSubcoreMesh(core_axis_name="c", subcore_axis_name="s"),
        scratch_types=[
            pltpu.VMEM((B_PER_W,), jnp.int32),
            pltpu.VMEM((B_PER_W,), jnp.int32),
            pltpu.VMEM((B_PER_W, DIM_MLP), jnp.float32),
            pltpu.VMEM((B_PER_W, DIM_MLP), jnp.float32),
            pltpu.VMEM((B_PER_W, DIM_GMF), jnp.float32),
            pltpu.VMEM((B_PER_W, DIM_GMF), jnp.float32),
            pltpu.SemaphoreType.DMA,
            pltpu.SemaphoreType.DMA,
            pltpu.SemaphoreType.DMA,
            pltpu.SemaphoreType.DMA,
        ],
        compiler_params=pltpu.CompilerParams(use_tc_tiling_on_sc=False),
    )(_sc_gather_body)


def _tc_mlp_body(um, im, uf, if_, w1, b1, g1, be1, w2, b2, g2, be2, wout, out):
    w1_ = w1[...]
    nt = (((1,), (1,)), ((), ()))
    h = lax.dot_general(um[...], w1_[:, :DIM_MLP], nt,
                        preferred_element_type=jnp.float32)
    h = h + lax.dot_general(im[...], w1_[:, DIM_MLP:], nt,
                            preferred_element_type=jnp.float32)
    h = jnp.maximum(h + b1[...], 0.0)
    mean = jnp.mean(h, axis=0, keepdims=True)
    var = jnp.mean((h - mean) ** 2, axis=0, keepdims=True)
    h = (h - mean) * lax.rsqrt(var + EPS) * g1[...] + be1[...]
    h = lax.dot_general(h, w2[...], nt, preferred_element_type=jnp.float32)
    h = jnp.maximum(h + b2[...], 0.0)
    mean = jnp.mean(h, axis=0, keepdims=True)
    var = jnp.mean((h - mean) ** 2, axis=0, keepdims=True)
    h = (h - mean) * lax.rsqrt(var + EPS) * g2[...] + be2[...]
    mf = uf[...] * if_[...]
    wout_ = wout[...]
    logits = (jnp.sum(h * wout_[:, :16], axis=1, keepdims=True)
              + jnp.sum(mf * wout_[:, 16:], axis=1, keepdims=True))
    out[...] = jax.nn.sigmoid(logits)


def kernel(user_indices, item_indices, E_user_mlp, E_item_mlp, E_user_mf,
           E_item_mf, W1, b1, g1, be1, W2, b2, g2, be2, Wout):
    u_idx = user_indices.astype(jnp.int32)
    i_idx = item_indices.astype(jnp.int32)
    pu_mlp, pi_mlp, pu_mf, pi_mf = _repack(
        E_user_mlp.T, E_item_mlp.T, E_user_mf.T, E_item_mf.T)
    um, im, uf, if_ = _make_sc_gather()(
        u_idx, i_idx,
        pu_mlp.reshape(NUM_ROWS, DIM_MLP), pi_mlp.reshape(NUM_ROWS, DIM_MLP),
        pu_mf.reshape(NUM_ROWS, DIM_GMF), pi_mf.reshape(NUM_ROWS, DIM_GMF))
    out = pl.pallas_call(
        _tc_mlp_body,
        out_shape=jax.ShapeDtypeStruct((BATCH, 1), jnp.float32),
    )(um, im, uf, if_,
      W1, b1.reshape(1, -1), g1.reshape(1, -1), be1.reshape(1, -1),
      W2, b2.reshape(1, -1), g2.reshape(1, -1), be2.reshape(1, -1), Wout)
    return out


# speedup vs baseline: 3.7259x; 2.9491x over previous
import functools

import jax
import jax.numpy as jnp
from jax import lax
from jax.experimental import pallas as pl
from jax.experimental.pallas import tpu as pltpu
from jax.experimental.pallas import tpu_sc as plsc

BATCH = 16384
NUM_ROWS = 1000000
DIM_MLP = 32
DIM_GMF = 16
EPS = 1e-5

NUM_CORES = 2
NUM_SUBCORES = 16
NUM_WORKERS = NUM_CORES * NUM_SUBCORES
B_PER_W = BATCH // NUM_WORKERS
LANES = 16
EXTRACT = True

BLK = 2048
SEG_MLP = 251904
SEG_GMF = 126976
GRID_MLP = SEG_MLP // BLK
GRID_GMF = SEG_GMF // BLK
N_SEG_MLP = 128 // DIM_MLP
N_SEG_GMF = 128 // DIM_GMF
IN_BLOCKS = (NUM_ROWS + BLK - 1) // BLK


def _repack_mlp_body(*refs):
    ins, outs = refs[:8], refs[8:]
    for t in range(2):
        x = jnp.concatenate([ins[4 * t + p][...] for p in range(4)], axis=0)
        outs[t][...] = x.T


def _repack_gmf_body(*refs):
    ins, outs = refs[:16], refs[16:]
    for t in range(2):
        x = jnp.concatenate([ins[8 * t + p][...] for p in range(8)], axis=0)
        outs[t][...] = x.T


def _repack_mlp(eu_t, ei_t):
    def in_spec(p):
        return pl.BlockSpec(
            (DIM_MLP, BLK),
            lambda g, p=p: (0, jnp.minimum(GRID_MLP * p + g, IN_BLOCKS - 1)))

    out_spec = pl.BlockSpec((BLK, 128), lambda g: (g, 0))
    return pl.pallas_call(
        _repack_mlp_body,
        grid=(GRID_MLP,),
        in_specs=[in_spec(p) for p in range(4)] * 2,
        out_specs=[out_spec, out_spec],
        out_shape=[jax.ShapeDtypeStruct((SEG_MLP, 128), jnp.float32)] * 2,
    )(*([eu_t] * 4 + [ei_t] * 4))


def _repack_gmf(eu_t, ei_t):
    def in_spec(p):
        return pl.BlockSpec(
            (DIM_GMF, BLK),
            lambda g, p=p: (0, jnp.minimum(GRID_GMF * p + g, IN_BLOCKS - 1)))

    out_spec = pl.BlockSpec((BLK, 128), lambda g: (g, 0))
    return pl.pallas_call(
        _repack_gmf_body,
        grid=(GRID_GMF,),
        in_specs=[in_spec(p) for p in range(8)] * 2,
        out_specs=[out_spec, out_spec],
        out_shape=[jax.ShapeDtypeStruct((SEG_GMF, 128), jnp.float32)] * 2,
    )(*([eu_t] * 8 + [ei_t] * 8))


def _sc_gather_body(u_idx_hbm, i_idx_hbm, qu_mlp, qi_mlp, qu_mf, qi_mf,
                    u_mlp_out, i_mlp_out, u_mf_out, i_mf_out,
                    uidx_v, iidx_v, s0_v, s1_v, s2_v, s3_v, p_v, gbuf,
                    um_v, im_v, uf_v, if_v, sem):
    wid = lax.axis_index("s") * NUM_CORES + lax.axis_index("c")
    base = wid * B_PER_W
    sl = pl.ds(base, B_PER_W)
    pltpu.sync_copy(u_idx_hbm.at[sl], uidx_v)
    pltpu.sync_copy(i_idx_hbm.at[sl], iidx_v)

    iota16 = lax.iota(jnp.int32, LANES)

    def seg_split(idx_ref, seg, n_seg, s_ref, p_off):
        def body(k, _):
            off = pl.multiple_of(k * LANES, LANES)
            u = idx_ref[pl.ds(off, LANES)]
            p = jnp.zeros((LANES,), jnp.int32)
            ones = jnp.ones((LANES,), jnp.int32)
            zeros = jnp.zeros((LANES,), jnp.int32)
            for m in range(1, n_seg):
                p = p + lax.select(u >= m * seg, ones, zeros)
            s_ref[pl.ds(off, LANES)] = u - p * seg
            p_v[pl.ds(p_off + off, LANES)] = p
            return _

        lax.fori_loop(0, B_PER_W // LANES, body, None)

    seg_split(uidx_v, SEG_MLP, N_SEG_MLP, s0_v, 0)
    seg_split(iidx_v, SEG_MLP, N_SEG_MLP, s1_v, B_PER_W)
    seg_split(uidx_v, SEG_GMF, N_SEG_GMF, s2_v, 2 * B_PER_W)
    seg_split(iidx_v, SEG_GMF, N_SEG_GMF, s3_v, 3 * B_PER_W)

    def gather_extract(q, dim, s_ref, s_off, out_ref):
        pltpu.async_copy(q.at[s_ref], gbuf, sem).wait()

        def body(j, _):
            jbase = pl.multiple_of(j * LANES, LANES)
            rows = jbase + iota16
            colbase = p_v[pl.ds(s_off + jbase, LANES)] * dim

            def inner(c, _):
                vals = plsc.load_gather(gbuf, [rows, colbase + c])
                plsc.store_scatter(out_ref, [rows, iota16 * 0 + c], vals)
                return _

            lax.fori_loop(0, dim, inner, None)
            return _

        if EXTRACT:
            lax.fori_loop(0, B_PER_W // LANES, body, None)

    gather_extract(qu_mlp, DIM_MLP, s0_v, 0, um_v)
    gather_extract(qi_mlp, DIM_MLP, s1_v, B_PER_W, im_v)
    gather_extract(qu_mf, DIM_GMF, s2_v, 2 * B_PER_W, uf_v)
    gather_extract(qi_mf, DIM_GMF, s3_v, 3 * B_PER_W, if_v)

    pltpu.sync_copy(um_v, u_mlp_out.at[sl])
    pltpu.sync_copy(im_v, i_mlp_out.at[sl])
    pltpu.sync_copy(uf_v, u_mf_out.at[sl])
    pltpu.sync_copy(if_v, i_mf_out.at[sl])


def _make_sc_gather():
    return functools.partial(
        pl.kernel,
        out_type=(
            jax.ShapeDtypeStruct((BATCH, DIM_MLP), jnp.float32),
            jax.ShapeDtypeStruct((BATCH, DIM_MLP), jnp.float32),
            jax.ShapeDtypeStruct((BATCH, DIM_GMF), jnp.float32),
            jax.ShapeDtypeStruct((BATCH, DIM_GMF), jnp.float32),
        ),
        mesh=plsc.VectorSubcoreMesh(core_axis_name="c", subcore_axis_name="s"),
        scratch_types=[
            pltpu.VMEM((B_PER_W,), jnp.int32),
            pltpu.VMEM((B_PER_W,), jnp.int32),
            pltpu.VMEM((B_PER_W,), jnp.int32),
            pltpu.VMEM((B_PER_W,), jnp.int32),
            pltpu.VMEM((B_PER_W,), jnp.int32),
            pltpu.VMEM((B_PER_W,), jnp.int32),
            pltpu.VMEM((4 * B_PER_W,), jnp.int32),
            pltpu.VMEM((B_PER_W, 128), jnp.float32),
            pltpu.VMEM((B_PER_W, DIM_MLP), jnp.float32),
            pltpu.VMEM((B_PER_W, DIM_MLP), jnp.float32),
            pltpu.VMEM((B_PER_W, DIM_GMF), jnp.float32),
            pltpu.VMEM((B_PER_W, DIM_GMF), jnp.float32),
            pltpu.SemaphoreType.DMA,
        ],
        compiler_params=pltpu.CompilerParams(use_tc_tiling_on_sc=False,
                                             needs_layout_passes=False),
    )(_sc_gather_body)


def _tc_mlp_body(um, im, uf, if_, w1, b1, g1, be1, w2, b2, g2, be2, wout, out):
    w1_ = w1[...]
    nt = (((1,), (1,)), ((), ()))
    h = lax.dot_general(um[...], w1_[:, :DIM_MLP], nt,
                        preferred_element_type=jnp.float32)
    h = h + lax.dot_general(im[...], w1_[:, DIM_MLP:], nt,
                            preferred_element_type=jnp.float32)
    h = jnp.maximum(h + b1[...], 0.0)
    mean = jnp.mean(h, axis=0, keepdims=True)
    var = jnp.mean((h - mean) ** 2, axis=0, keepdims=True)
    h = (h - mean) * lax.rsqrt(var + EPS) * g1[...] + be1[...]
    h = lax.dot_general(h, w2[...], nt, preferred_element_type=jnp.float32)
    h = jnp.maximum(h + b2[...], 0.0)
    mean = jnp.mean(h, axis=0, keepdims=True)
    var = jnp.mean((h - mean) ** 2, axis=0, keepdims=True)
    h = (h - mean) * lax.rsqrt(var + EPS) * g2[...] + be2[...]
    mf = uf[...] * if_[...]
    wout_ = wout[...]
    logits = (jnp.sum(h * wout_[:, :16], axis=1, keepdims=True)
              + jnp.sum(mf * wout_[:, 16:], axis=1, keepdims=True))
    out[...] = jax.nn.sigmoid(logits)


def kernel(user_indices, item_indices, E_user_mlp, E_item_mlp, E_user_mf,
           E_item_mf, W1, b1, g1, be1, W2, b2, g2, be2, Wout):
    u_idx = user_indices.astype(jnp.int32)
    i_idx = item_indices.astype(jnp.int32)
    qu_mlp, qi_mlp = _repack_mlp(E_user_mlp.T, E_item_mlp.T)
    qu_mf, qi_mf = _repack_gmf(E_user_mf.T, E_item_mf.T)
    um, im, uf, if_ = _make_sc_gather()(
        u_idx, i_idx, qu_mlp, qi_mlp, qu_mf, qi_mf)
    out = pl.pallas_call(
        _tc_mlp_body,
        out_shape=jax.ShapeDtypeStruct((BATCH, 1), jnp.float32),
    )(um, im, uf, if_,
      W1, b1.reshape(1, -1), g1.reshape(1, -1), be1.reshape(1, -1),
      W2, b2.reshape(1, -1), g2.reshape(1, -1), be2.reshape(1, -1), Wout)
    return out


# speedup vs baseline: 4.8101x; 1.2910x over previous
import functools

import jax
import jax.numpy as jnp
from jax import lax
from jax.experimental import pallas as pl
from jax.experimental.pallas import tpu as pltpu
from jax.experimental.pallas import tpu_sc as plsc

BATCH = 16384
NUM_ROWS = 1000000
DIM_MLP = 32
DIM_GMF = 16
EPS = 1e-5

NUM_CORES = 2
NUM_SUBCORES = 16
NUM_WORKERS = NUM_CORES * NUM_SUBCORES
B_PER_W = BATCH // NUM_WORKERS
LANES = 16

BLK = 8192
SEG_MLP = 262144
SEG_GMF = 131072
GRID_MLP = SEG_MLP // BLK
GRID_GMF = SEG_GMF // BLK
N_SEG_MLP = 128 // DIM_MLP
N_SEG_GMF = 128 // DIM_GMF
IN_BLOCKS = (NUM_ROWS + BLK - 1) // BLK


def _repack_mlp_body(*refs):
    ins, outs = refs[:8], refs[8:]
    for t in range(2):
        x = jnp.concatenate([ins[4 * t + p][...] for p in range(4)], axis=0)
        outs[t][...] = x.T


def _repack_gmf_body(*refs):
    ins, outs = refs[:16], refs[16:]
    for t in range(2):
        x = jnp.concatenate([ins[8 * t + p][...] for p in range(8)], axis=0)
        outs[t][...] = x.T


def _repack_mlp(eu_t, ei_t):
    def in_spec(p):
        return pl.BlockSpec(
            (DIM_MLP, BLK),
            lambda g, p=p: (0, jnp.minimum(GRID_MLP * p + g, IN_BLOCKS - 1)))

    out_spec = pl.BlockSpec((BLK, 128), lambda g: (g, 0))
    return pl.pallas_call(
        _repack_mlp_body,
        grid=(GRID_MLP,),
        in_specs=[in_spec(p) for p in range(4)] * 2,
        out_specs=[out_spec, out_spec],
        out_shape=[jax.ShapeDtypeStruct((SEG_MLP, 128), jnp.float32)] * 2,
    )(*([eu_t] * 4 + [ei_t] * 4))


def _repack_gmf(eu_t, ei_t):
    def in_spec(p):
        return pl.BlockSpec(
            (DIM_GMF, BLK),
            lambda g, p=p: (0, jnp.minimum(GRID_GMF * p + g, IN_BLOCKS - 1)))

    out_spec = pl.BlockSpec((BLK, 128), lambda g: (g, 0))
    return pl.pallas_call(
        _repack_gmf_body,
        grid=(GRID_GMF,),
        in_specs=[in_spec(p) for p in range(8)] * 2,
        out_specs=[out_spec, out_spec],
        out_shape=[jax.ShapeDtypeStruct((SEG_GMF, 128), jnp.float32)] * 2,
    )(*([eu_t] * 8 + [ei_t] * 8))


def _sc_pair_body(seg, n_seg, dim, u_idx_hbm, i_idx_hbm, qu, qi,
                  u_out, i_out, uidx_v, iidx_v, s0_v, s1_v, p_v, gbuf,
                  u_v, i_v, sem):
    wid = lax.axis_index("s") * NUM_CORES + lax.axis_index("c")
    base = wid * B_PER_W
    sl = pl.ds(base, B_PER_W)
    pltpu.sync_copy(u_idx_hbm.at[sl], uidx_v)
    pltpu.sync_copy(i_idx_hbm.at[sl], iidx_v)

    iota16 = lax.iota(jnp.int32, LANES)

    def seg_split(idx_ref, s_ref, p_off):
        def body(k, _):
            off = pl.multiple_of(k * LANES, LANES)
            u = idx_ref[pl.ds(off, LANES)]
            p = jnp.zeros((LANES,), jnp.int32)
            ones = jnp.ones((LANES,), jnp.int32)
            zeros = jnp.zeros((LANES,), jnp.int32)
            for m in range(1, n_seg):
                p = p + lax.select(u >= m * seg, ones, zeros)
            s_ref[pl.ds(off, LANES)] = u - p * seg
            p_v[pl.ds(p_off + off, LANES)] = p
            return _

        lax.fori_loop(0, B_PER_W // LANES, body, None)

    seg_split(uidx_v, s0_v, 0)
    seg_split(iidx_v, s1_v, B_PER_W)

    def gather_extract(q, s_ref, s_off, out_ref):
        pltpu.async_copy(q.at[s_ref], gbuf, sem).wait()

        def body(j, _):
            jbase = pl.multiple_of(j * LANES, LANES)
            rows = jbase + iota16
            colbase = p_v[pl.ds(s_off + jbase, LANES)] * dim

            for c in range(dim):
                vals = plsc.load_gather(gbuf, [rows, colbase + c])
                plsc.store_scatter(out_ref, [rows, iota16 * 0 + c], vals)
            return _

        lax.fori_loop(0, B_PER_W // LANES, body, None)

    gather_extract(qu, s0_v, 0, u_v)
    gather_extract(qi, s1_v, B_PER_W, i_v)

    pltpu.sync_copy(u_v, u_out.at[sl])
    pltpu.sync_copy(i_v, i_out.at[sl])


def _make_sc_gather(seg, n_seg, dim):
    return functools.partial(
        pl.kernel,
        out_type=(
            jax.ShapeDtypeStruct((BATCH, dim), jnp.float32),
            jax.ShapeDtypeStruct((BATCH, dim), jnp.float32),
        ),
        mesh=plsc.VectorSubcoreMesh(core_axis_name="c", subcore_axis_name="s"),
        scratch_types=[
            pltpu.VMEM((B_PER_W,), jnp.int32),
            pltpu.VMEM((B_PER_W,), jnp.int32),
            pltpu.VMEM((B_PER_W,), jnp.int32),
            pltpu.VMEM((B_PER_W,), jnp.int32),
            pltpu.VMEM((2 * B_PER_W,), jnp.int32),
            pltpu.VMEM((B_PER_W, 128), jnp.float32),
            pltpu.VMEM((B_PER_W, dim), jnp.float32),
            pltpu.VMEM((B_PER_W, dim), jnp.float32),
            pltpu.SemaphoreType.DMA,
        ],
        compiler_params=pltpu.CompilerParams(use_tc_tiling_on_sc=False,
                                             needs_layout_passes=False),
    )(functools.partial(_sc_pair_body, seg, n_seg, dim))


def _tc_mlp_body(um, im, uf, if_, w1, b1, g1, be1, w2, b2, g2, be2, wout, out):
    w1_ = w1[...]
    nt = (((1,), (1,)), ((), ()))
    h = lax.dot_general(um[...], w1_[:, :DIM_MLP], nt,
                        preferred_element_type=jnp.float32)
    h = h + lax.dot_general(im[...], w1_[:, DIM_MLP:], nt,
                            preferred_element_type=jnp.float32)
    h = jnp.maximum(h + b1[...], 0.0)
    mean = jnp.mean(h, axis=0, keepdims=True)
    var = jnp.mean((h - mean) ** 2, axis=0, keepdims=True)
    h = (h - mean) * lax.rsqrt(var + EPS) * g1[...] + be1[...]
    h = lax.dot_general(h, w2[...], nt, preferred_element_type=jnp.float32)
    h = jnp.maximum(h + b2[...], 0.0)
    mean = jnp.mean(h, axis=0, keepdims=True)
    var = jnp.mean((h - mean) ** 2, axis=0, keepdims=True)
    h = (h - mean) * lax.rsqrt(var + EPS) * g2[...] + be2[...]
    mf = uf[...] * if_[...]
    wout_ = wout[...]
    logits = (jnp.sum(h * wout_[:, :16], axis=1, keepdims=True)
              + jnp.sum(mf * wout_[:, 16:], axis=1, keepdims=True))
    out[...] = jax.nn.sigmoid(logits)


def kernel(user_indices, item_indices, E_user_mlp, E_item_mlp, E_user_mf,
           E_item_mf, W1, b1, g1, be1, W2, b2, g2, be2, Wout):
    u_idx = user_indices.astype(jnp.int32)
    i_idx = item_indices.astype(jnp.int32)
    qu_mlp, qi_mlp = _repack_mlp(E_user_mlp.T, E_item_mlp.T)
    um, im = _make_sc_gather(SEG_MLP, N_SEG_MLP, DIM_MLP)(
        u_idx, i_idx, qu_mlp, qi_mlp)
    qu_mf, qi_mf = _repack_gmf(E_user_mf.T, E_item_mf.T)
    uf, if_ = _make_sc_gather(SEG_GMF, N_SEG_GMF, DIM_GMF)(
        u_idx, i_idx, qu_mf, qi_mf)
    out = pl.pallas_call(
        _tc_mlp_body,
        out_shape=jax.ShapeDtypeStruct((BATCH, 1), jnp.float32),
    )(um, im, uf, if_,
      W1, b1.reshape(1, -1), g1.reshape(1, -1), be1.reshape(1, -1),
      W2, b2.reshape(1, -1), g2.reshape(1, -1), be2.reshape(1, -1), Wout)
    return out


# speedup vs baseline: 4.8551x; 1.0093x over previous
import functools

import jax
import jax.numpy as jnp
from jax import lax
from jax.experimental import pallas as pl
from jax.experimental.pallas import tpu as pltpu
from jax.experimental.pallas import tpu_sc as plsc

BATCH = 16384
NUM_ROWS = 1000000
DIM_MLP = 32
DIM_GMF = 16
EPS = 1e-5

NUM_CORES = 2
NUM_SUBCORES = 16
NUM_WORKERS = NUM_CORES * NUM_SUBCORES
B_PER_W = BATCH // NUM_WORKERS
LANES = 16

BLK = 12288
SEG_MLP = 258048
SEG_GMF = 135168
GRID_MLP = SEG_MLP // BLK
GRID_GMF = SEG_GMF // BLK
N_SEG_MLP = 128 // DIM_MLP
N_SEG_GMF = 128 // DIM_GMF
IN_BLOCKS = (NUM_ROWS + BLK - 1) // BLK


def _repack_mlp_body(*refs):
    ins, outs = refs[:8], refs[8:]
    for t in range(2):
        x = jnp.concatenate([ins[4 * t + p][...] for p in range(4)], axis=0)
        outs[t][...] = x.T


def _repack_gmf_body(*refs):
    ins, outs = refs[:16], refs[16:]
    for t in range(2):
        x = jnp.concatenate([ins[8 * t + p][...] for p in range(8)], axis=0)
        outs[t][...] = x.T


def _repack_mlp(eu_t, ei_t):
    def in_spec(p):
        return pl.BlockSpec(
            (DIM_MLP, BLK),
            lambda g, p=p: (0, jnp.minimum(GRID_MLP * p + g, IN_BLOCKS - 1)))

    out_spec = pl.BlockSpec((BLK, 128), lambda g: (g, 0))
    return pl.pallas_call(
        _repack_mlp_body,
        grid=(GRID_MLP,),
        in_specs=[in_spec(p) for p in range(4)] * 2,
        out_specs=[out_spec, out_spec],
        out_shape=[jax.ShapeDtypeStruct((SEG_MLP, 128), jnp.float32)] * 2,
        compiler_params=pltpu.CompilerParams(
            vmem_limit_bytes=110 * 1024 * 1024),
    )(*([eu_t] * 4 + [ei_t] * 4))


def _repack_gmf(eu_t, ei_t):
    def in_spec(p):
        return pl.BlockSpec(
            (DIM_GMF, BLK),
            lambda g, p=p: (0, jnp.minimum(GRID_GMF * p + g, IN_BLOCKS - 1)))

    out_spec = pl.BlockSpec((BLK, 128), lambda g: (g, 0))
    return pl.pallas_call(
        _repack_gmf_body,
        grid=(GRID_GMF,),
        in_specs=[in_spec(p) for p in range(8)] * 2,
        out_specs=[out_spec, out_spec],
        out_shape=[jax.ShapeDtypeStruct((SEG_GMF, 128), jnp.float32)] * 2,
        compiler_params=pltpu.CompilerParams(
            vmem_limit_bytes=110 * 1024 * 1024),
    )(*([eu_t] * 8 + [ei_t] * 8))


def _sc_pair_body(seg, n_seg, dim, u_idx_hbm, i_idx_hbm, qu, qi,
                  u_out, i_out, uidx_v, iidx_v, s0_v, s1_v, p_v, gbuf,
                  u_v, i_v, sem):
    wid = lax.axis_index("s") * NUM_CORES + lax.axis_index("c")
    base = wid * B_PER_W
    sl = pl.ds(base, B_PER_W)
    pltpu.sync_copy(u_idx_hbm.at[sl], uidx_v)
    pltpu.sync_copy(i_idx_hbm.at[sl], iidx_v)

    iota16 = lax.iota(jnp.int32, LANES)

    def seg_split(idx_ref, s_ref, p_off):
        def body(k, _):
            off = pl.multiple_of(k * LANES, LANES)
            u = idx_ref[pl.ds(off, LANES)]
            p = jnp.zeros((LANES,), jnp.int32)
            ones = jnp.ones((LANES,), jnp.int32)
            zeros = jnp.zeros((LANES,), jnp.int32)
            for m in range(1, n_seg):
                p = p + lax.select(u >= m * seg, ones, zeros)
            s_ref[pl.ds(off, LANES)] = u - p * seg
            p_v[pl.ds(p_off + off, LANES)] = p
            return _

        lax.fori_loop(0, B_PER_W // LANES, body, None)

    seg_split(uidx_v, s0_v, 0)
    seg_split(iidx_v, s1_v, B_PER_W)

    def gather_extract(q, s_ref, s_off, out_ref):
        pltpu.async_copy(q.at[s_ref], gbuf, sem).wait()

        def body(j, _):
            jbase = pl.multiple_of(j * LANES, LANES)
            rows = jbase + iota16
            colbase = p_v[pl.ds(s_off + jbase, LANES)] * dim

            for c in range(dim):
                vals = plsc.load_gather(gbuf, [rows, colbase + c])
                plsc.store_scatter(out_ref, [rows, iota16 * 0 + c], vals)
            return _

        lax.fori_loop(0, B_PER_W // LANES, body, None)

    gather_extract(qu, s0_v, 0, u_v)
    gather_extract(qi, s1_v, B_PER_W, i_v)

    pltpu.sync_copy(u_v, u_out.at[sl])
    pltpu.sync_copy(i_v, i_out.at[sl])


def _make_sc_gather(seg, n_seg, dim):
    return functools.partial(
        pl.kernel,
        out_type=(
            jax.ShapeDtypeStruct((BATCH, dim), jnp.float32),
            jax.ShapeDtypeStruct((BATCH, dim), jnp.float32),
        ),
        mesh=plsc.VectorSubcoreMesh(core_axis_name="c", subcore_axis_name="s"),
        scratch_types=[
            pltpu.VMEM((B_PER_W,), jnp.int32),
            pltpu.VMEM((B_PER_W,), jnp.int32),
            pltpu.VMEM((B_PER_W,), jnp.int32),
            pltpu.VMEM((B_PER_W,), jnp.int32),
            pltpu.VMEM((2 * B_PER_W,), jnp.int32),
            pltpu.VMEM((B_PER_W, 128), jnp.float32),
            pltpu.VMEM((B_PER_W, dim), jnp.float32),
            pltpu.VMEM((B_PER_W, dim), jnp.float32),
            pltpu.SemaphoreType.DMA,
        ],
        compiler_params=pltpu.CompilerParams(use_tc_tiling_on_sc=False,
                                             needs_layout_passes=False),
    )(functools.partial(_sc_pair_body, seg, n_seg, dim))


def _tc_mlp_body(um, im, uf, if_, w1, b1, g1, be1, w2, b2, g2, be2, wout, out):
    w1_ = w1[...]
    nt = (((1,), (1,)), ((), ()))
    h = lax.dot_general(um[...], w1_[:, :DIM_MLP], nt,
                        preferred_element_type=jnp.float32)
    h = h + lax.dot_general(im[...], w1_[:, DIM_MLP:], nt,
                            preferred_element_type=jnp.float32)
    h = jnp.maximum(h + b1[...], 0.0)
    mean = jnp.mean(h, axis=0, keepdims=True)
    var = jnp.mean((h - mean) ** 2, axis=0, keepdims=True)
    h = (h - mean) * lax.rsqrt(var + EPS) * g1[...] + be1[...]
    h = lax.dot_general(h, w2[...], nt, preferred_element_type=jnp.float32)
    h = jnp.maximum(h + b2[...], 0.0)
    mean = jnp.mean(h, axis=0, keepdims=True)
    var = jnp.mean((h - mean) ** 2, axis=0, keepdims=True)
    h = (h - mean) * lax.rsqrt(var + EPS) * g2[...] + be2[...]
    mf = uf[...] * if_[...]
    wout_ = wout[...]
    logits = (jnp.sum(h * wout_[:, :16], axis=1, keepdims=True)
              + jnp.sum(mf * wout_[:, 16:], axis=1, keepdims=True))
    out[...] = jax.nn.sigmoid(logits)


def kernel(user_indices, item_indices, E_user_mlp, E_item_mlp, E_user_mf,
           E_item_mf, W1, b1, g1, be1, W2, b2, g2, be2, Wout):
    u_idx = user_indices.astype(jnp.int32)
    i_idx = item_indices.astype(jnp.int32)
    qu_mlp, qi_mlp = _repack_mlp(E_user_mlp.T, E_item_mlp.T)
    um, im = _make_sc_gather(SEG_MLP, N_SEG_MLP, DIM_MLP)(
        u_idx, i_idx, qu_mlp, qi_mlp)
    qu_mf, qi_mf = _repack_gmf(E_user_mf.T, E_item_mf.T)
    uf, if_ = _make_sc_gather(SEG_GMF, N_SEG_GMF, DIM_GMF)(
        u_idx, i_idx, qu_mf, qi_mf)
    out = pl.pallas_call(
        _tc_mlp_body,
        out_shape=jax.ShapeDtypeStruct((BATCH, 1), jnp.float32),
    )(um, im, uf, if_,
      W1, b1.reshape(1, -1), g1.reshape(1, -1), be1.reshape(1, -1),
      W2, b2.reshape(1, -1), g2.reshape(1, -1), be2.reshape(1, -1), Wout)
    return out
